# baseline (speedup 1.0000x reference)
import functools

import jax
import jax.numpy as jnp
from jax import lax
from jax.experimental import pallas as pl
from jax.experimental.pallas import tpu as pltpu
from jax.experimental.pallas import tpu_sc as plsc

NC = 2
NS = 16
NW = NC * NS
K = 128


def _sc_agg_kernel(n_acc, feat, chunks):
    rpt = n_acc // NS

    mesh = plsc.VectorSubcoreMesh(core_axis_name="c", subcore_axis_name="s")

    HK = K // 2

    @functools.partial(
        pl.kernel,
        out_type=jax.ShapeDtypeStruct((NC, n_acc, feat), jnp.float32),
        mesh=mesh,
        scratch_types=[
            pltpu.VMEM((chunks, K), jnp.int32),
            pltpu.VMEM((chunks, K), jnp.int32),
            pltpu.VMEM((HK, feat), jnp.float32),
            pltpu.VMEM((HK, feat), jnp.float32),
            pltpu.VMEM_SHARED((n_acc, feat), jnp.float32),
            pltpu.SemaphoreType.DMA,
            pltpu.SemaphoreType.DMA,
            pltpu.SemaphoreType.DMA,
            pltpu.SemaphoreType.DMA,
        ])
    def agg(table_hbm, srcB_hbm, dstB_hbm, zf_hbm,
            pout_hbm, src_v, dst_v, ga, gb, acc_sh,
            sem_a, sem_b, ssem_a, ssem_b):
        c = lax.axis_index("c")
        s = lax.axis_index("s")
        wid = c * NS + s

        pltpu.sync_copy(zf_hbm.at[pl.ds(s * rpt, rpt)],
                        acc_sh.at[pl.ds(s * rpt, rpt)])
        pltpu.sync_copy(srcB_hbm.at[wid], src_v)
        pltpu.sync_copy(dstB_hbm.at[wid], dst_v)
        plsc.subcore_barrier()

        def sidx(i, h):
            return src_v.at[i, pl.ds(h * HK, HK)]

        def didx(i, h):
            return dst_v.at[i, pl.ds(h * HK, HK)]

        def start(i, h, buf, sem):
            pltpu.async_copy(table_hbm.at[sidx(i, h)], buf, sem)

        def wait(i, h, buf, sem):
            pltpu.make_async_copy(table_hbm.at[sidx(i, h)], buf, sem).wait()

        def scat_start(i, h, buf, sem):
            pltpu.async_copy(buf, acc_sh.at[didx(i, h)], sem, add=True)

        def scat_wait(i, h, buf, sem):
            pltpu.make_async_copy(buf, acc_sh.at[didx(i, h)], sem).wait()

        start(0, 0, ga, sem_a)

        def step(i, carry):
            start(i, 1, gb, sem_b)
            wait(i, 0, ga, sem_a)
            scat_start(i, 0, ga, ssem_a)
            wait(i, 1, gb, sem_b)
            scat_start(i, 1, gb, ssem_b)
            scat_wait(i, 0, ga, ssem_a)

            @pl.when(i + 1 < chunks)
            def _():
                start(i + 1, 0, ga, sem_a)

            scat_wait(i, 1, gb, ssem_b)
            return carry

        lax.fori_loop(0, chunks, step, 0)
        plsc.subcore_barrier()

        pltpu.sync_copy(acc_sh.at[pl.ds(s * rpt, rpt)],
                        pout_hbm.at[c, pl.ds(s * rpt, rpt)])

    return agg


def _sc_cnt_kernel(n_acc, feat, chunks):
    rpt = n_acc // NS
    mesh = plsc.VectorSubcoreMesh(core_axis_name="c", subcore_axis_name="s")

    @functools.partial(
        pl.kernel,
        out_type=jax.ShapeDtypeStruct((NC, n_acc, feat), jnp.float32),
        mesh=mesh,
        scratch_types=[
            pltpu.VMEM((chunks, K), jnp.int32),
            pltpu.VMEM((K, feat), jnp.float32),
            pltpu.VMEM_SHARED((n_acc, feat), jnp.float32),
            pltpu.SemaphoreType.DMA,
            pltpu.SemaphoreType.DMA,
        ])
    def cntk(dstB_hbm, ones_hbm, zf_hbm, cnt_hbm, dst_v, ones_v, cnt_sh,
             sem_a, sem_b):
        c = lax.axis_index("c")
        s = lax.axis_index("s")
        wid = c * NS + s
        pltpu.sync_copy(zf_hbm.at[pl.ds(s * rpt, rpt)],
                        cnt_sh.at[pl.ds(s * rpt, rpt)])
        pltpu.sync_copy(dstB_hbm.at[wid], dst_v)
        pltpu.sync_copy(ones_hbm, ones_v)
        plsc.subcore_barrier()

        def start(j, sem):
            pltpu.async_copy(ones_v, cnt_sh.at[dst_v.at[j]], sem, add=True)

        def wait(j, sem):
            pltpu.make_async_copy(ones_v, cnt_sh.at[dst_v.at[j]], sem).wait()

        start(0, sem_a)

        def step(i, carry):
            j0 = 2 * i
            j1 = j0 + 1

            @pl.when(j1 < chunks)
            def _():
                start(j1, sem_b)

            wait(j0, sem_a)

            @pl.when(j0 + 2 < chunks)
            def _():
                start(j0 + 2, sem_a)

            @pl.when(j1 < chunks)
            def _():
                wait(j1, sem_b)

            return carry

        lax.fori_loop(0, (chunks + 1) // 2, step, 0)
        plsc.subcore_barrier()
        pltpu.sync_copy(cnt_sh.at[pl.ds(s * rpt, rpt)],
                        cnt_hbm.at[c, pl.ds(s * rpt, rpt)])

    return cntk


_DN = (((1,), (1,)), ((), ()))


def _tc_self(h, Wr, bl):
    n, feat = h.shape
    bn = 1000

    def body(h_ref, wr_ref, bl_ref, o_ref):
        o_ref[...] = lax.dot_general(
            h_ref[...], wr_ref[...], _DN,
            preferred_element_type=jnp.float32,
            precision=lax.Precision.HIGHEST) + bl_ref[...]

    return pl.pallas_call(
        body,
        grid=(n // bn,),
        in_specs=[
            pl.BlockSpec((bn, feat), lambda i: (i, 0)),
            pl.BlockSpec((feat, feat), lambda i: (0, 0)),
            pl.BlockSpec((1, feat), lambda i: (0, 0)),
        ],
        out_specs=pl.BlockSpec((bn, feat), lambda i: (i, 0)),
        out_shape=jax.ShapeDtypeStruct((n, feat), jnp.float32),
    )(h, Wr, bl)


def _tc_mix(p, cnt, selfterm, Wl, relu):
    n, feat = selfterm.shape
    bn = 1000

    def body(p_ref, cnt_ref, st_ref, wl_ref, o_ref):
        psum = p_ref[0] + p_ref[1]
        c = cnt_ref[0][:, 0:1] + cnt_ref[1][:, 0:1]
        mean = psum * (1.0 / jnp.maximum(c, 1.0))
        acc = lax.dot_general(mean, wl_ref[...], _DN,
                              preferred_element_type=jnp.float32,
                              precision=lax.Precision.HIGHEST)
        acc = acc + st_ref[...]
        o_ref[...] = jnp.maximum(acc, 0.0) if relu else acc

    return pl.pallas_call(
        body,
        grid=(n // bn,),
        in_specs=[
            pl.BlockSpec((NC, bn, feat), lambda i: (0, i, 0)),
            pl.BlockSpec((NC, bn, feat), lambda i: (0, i, 0)),
            pl.BlockSpec((bn, feat), lambda i: (i, 0)),
            pl.BlockSpec((feat, feat), lambda i: (0, 0)),
        ],
        out_specs=pl.BlockSpec((bn, feat), lambda i: (i, 0)),
        out_shape=jax.ShapeDtypeStruct((n, feat), jnp.float32),
    )(p, cnt, selfterm, Wl)


def kernel(x, edge_index, W1l, b1l, W1r, W2l, b2l, W2r):
    n, feat = x.shape
    e = edge_index.shape[1]
    pad_rows = NS
    n_acc = -(-(n + pad_rows) // (NS * 8)) * (NS * 8)

    chunks = -(-e // (NW * K))
    e_pad = NW * chunks * K
    src = edge_index[0].astype(jnp.int32)
    dst = edge_index[1].astype(jnp.int32)
    if e_pad != e:
        pidx = jnp.arange(e_pad - e, dtype=jnp.int32)
        src = jnp.concatenate([src, pidx % n])
        dst = jnp.concatenate([dst, n + pidx % pad_rows])
    srcB = src.reshape(NW, chunks, K)
    dstB = dst.reshape(NW, chunks, K)

    ones = jnp.ones((K, feat), jnp.float32)
    zf = jnp.zeros((n_acc, feat), jnp.float32)

    agg = _sc_agg_kernel(n_acc, feat, chunks)
    cntk = _sc_cnt_kernel(n_acc, feat, chunks)

    cnt = cntk(dstB, ones, zf)
    self1 = _tc_self(x, W1r, b1l.reshape(1, feat))
    p1 = agg(x, srcB, dstB, zf)
    h = _tc_mix(p1, cnt, self1, W1l, relu=True)
    self2 = _tc_self(h, W2r, b2l.reshape(1, feat))
    p2 = agg(h, srcB, dstB, zf)
    z = _tc_mix(p2, cnt, self2, W2l, relu=False)
    return z


# speedup vs baseline: 1.1669x; 1.1669x over previous
import functools

import jax
import jax.numpy as jnp
from jax import lax
from jax.experimental import pallas as pl
from jax.experimental.pallas import tpu as pltpu
from jax.experimental.pallas import tpu_sc as plsc

NC = 2
NS = 16
NW = NC * NS
K = 128


def _sc_agg_kernel(n_acc, feat, chunks):
    rpt = n_acc // NS

    mesh = plsc.VectorSubcoreMesh(core_axis_name="c", subcore_axis_name="s")

    HK = K // 2

    @functools.partial(
        pl.kernel,
        out_type=jax.ShapeDtypeStruct((NC, n_acc, feat), jnp.float32),
        mesh=mesh,
        scratch_types=[
            pltpu.VMEM((chunks, K), jnp.int32),
            pltpu.VMEM((chunks, K), jnp.int32),
            pltpu.VMEM((HK, feat), jnp.float32),
            pltpu.VMEM((HK, feat), jnp.float32),
            pltpu.VMEM_SHARED((n_acc, feat), jnp.float32),
            pltpu.SemaphoreType.DMA,
            pltpu.SemaphoreType.DMA,
        ])
    def agg(table_hbm, srcB_hbm, dstB_hbm, zf_hbm,
            pout_hbm, src_v, dst_v, ga, gb, acc_sh, sem_a, sem_b):
        c = lax.axis_index("c")
        s = lax.axis_index("s")
        wid = c * NS + s

        pltpu.sync_copy(zf_hbm.at[pl.ds(s * rpt, rpt)],
                        acc_sh.at[pl.ds(s * rpt, rpt)])
        pltpu.sync_copy(srcB_hbm.at[wid], src_v)
        pltpu.sync_copy(dstB_hbm.at[wid], dst_v)
        plsc.subcore_barrier()

        def sidx(i, h):
            return src_v.at[i, pl.ds(h * HK, HK)]

        def didx(i, h):
            return dst_v.at[i, pl.ds(h * HK, HK)]

        def start(i, h, buf, sem):
            pltpu.async_copy(table_hbm.at[sidx(i, h)], buf, sem)

        def wait(i, h, buf, sem):
            pltpu.make_async_copy(table_hbm.at[sidx(i, h)], buf, sem).wait()

        def scat(i, h, buf):
            pltpu.sync_copy(buf, acc_sh.at[didx(i, h)], add=True)

        start(0, 0, ga, sem_a)

        def step(i, carry):
            start(i, 1, gb, sem_b)
            wait(i, 0, ga, sem_a)
            scat(i, 0, ga)

            @pl.when(i + 1 < chunks)
            def _():
                start(i + 1, 0, ga, sem_a)

            wait(i, 1, gb, sem_b)
            scat(i, 1, gb)
            return carry

        lax.fori_loop(0, chunks, step, 0)
        plsc.subcore_barrier()

        pltpu.sync_copy(acc_sh.at[pl.ds(s * rpt, rpt)],
                        pout_hbm.at[c, pl.ds(s * rpt, rpt)])

    return agg


def _sc_cnt_kernel(n_acc, feat, chunks):
    rpt = n_acc // NS
    mesh = plsc.VectorSubcoreMesh(core_axis_name="c", subcore_axis_name="s")

    @functools.partial(
        pl.kernel,
        out_type=jax.ShapeDtypeStruct((NC, n_acc, feat), jnp.float32),
        mesh=mesh,
        scratch_types=[
            pltpu.VMEM((chunks, K), jnp.int32),
            pltpu.VMEM((K, feat), jnp.float32),
            pltpu.VMEM_SHARED((n_acc, feat), jnp.float32),
            pltpu.SemaphoreType.DMA,
            pltpu.SemaphoreType.DMA,
        ])
    def cntk(dstB_hbm, ones_hbm, zf_hbm, cnt_hbm, dst_v, ones_v, cnt_sh,
             sem_a, sem_b):
        c = lax.axis_index("c")
        s = lax.axis_index("s")
        wid = c * NS + s
        pltpu.sync_copy(zf_hbm.at[pl.ds(s * rpt, rpt)],
                        cnt_sh.at[pl.ds(s * rpt, rpt)])
        pltpu.sync_copy(dstB_hbm.at[wid], dst_v)
        pltpu.sync_copy(ones_hbm, ones_v)
        plsc.subcore_barrier()

        def start(j, sem):
            pltpu.async_copy(ones_v, cnt_sh.at[dst_v.at[j]], sem, add=True)

        def wait(j, sem):
            pltpu.make_async_copy(ones_v, cnt_sh.at[dst_v.at[j]], sem).wait()

        start(0, sem_a)

        def step(i, carry):
            j0 = 2 * i
            j1 = j0 + 1

            @pl.when(j1 < chunks)
            def _():
                start(j1, sem_b)

            wait(j0, sem_a)

            @pl.when(j0 + 2 < chunks)
            def _():
                start(j0 + 2, sem_a)

            @pl.when(j1 < chunks)
            def _():
                wait(j1, sem_b)

            return carry

        lax.fori_loop(0, (chunks + 1) // 2, step, 0)
        plsc.subcore_barrier()
        pltpu.sync_copy(cnt_sh.at[pl.ds(s * rpt, rpt)],
                        cnt_hbm.at[c, pl.ds(s * rpt, rpt)])

    return cntk


_DN = (((1,), (1,)), ((), ()))


def _tc_self(h, Wr, bl):
    n, feat = h.shape
    bn = 1000

    def body(h_ref, wr_ref, bl_ref, o_ref):
        o_ref[...] = lax.dot_general(
            h_ref[...], wr_ref[...], _DN,
            preferred_element_type=jnp.float32,
            precision=lax.Precision.HIGHEST) + bl_ref[...]

    return pl.pallas_call(
        body,
        grid=(n // bn,),
        in_specs=[
            pl.BlockSpec((bn, feat), lambda i: (i, 0)),
            pl.BlockSpec((feat, feat), lambda i: (0, 0)),
            pl.BlockSpec((1, feat), lambda i: (0, 0)),
        ],
        out_specs=pl.BlockSpec((bn, feat), lambda i: (i, 0)),
        out_shape=jax.ShapeDtypeStruct((n, feat), jnp.float32),
    )(h, Wr, bl)


def _tc_mix(p, cnt, selfterm, Wl, relu):
    n, feat = selfterm.shape
    bn = 1000

    def body(p_ref, cnt_ref, st_ref, wl_ref, o_ref):
        psum = p_ref[0] + p_ref[1]
        c = cnt_ref[0][:, 0:1] + cnt_ref[1][:, 0:1]
        mean = psum * (1.0 / jnp.maximum(c, 1.0))
        acc = lax.dot_general(mean, wl_ref[...], _DN,
                              preferred_element_type=jnp.float32,
                              precision=lax.Precision.HIGHEST)
        acc = acc + st_ref[...]
        o_ref[...] = jnp.maximum(acc, 0.0) if relu else acc

    return pl.pallas_call(
        body,
        grid=(n // bn,),
        in_specs=[
            pl.BlockSpec((NC, bn, feat), lambda i: (0, i, 0)),
            pl.BlockSpec((NC, bn, feat), lambda i: (0, i, 0)),
            pl.BlockSpec((bn, feat), lambda i: (i, 0)),
            pl.BlockSpec((feat, feat), lambda i: (0, 0)),
        ],
        out_specs=pl.BlockSpec((bn, feat), lambda i: (i, 0)),
        out_shape=jax.ShapeDtypeStruct((n, feat), jnp.float32),
    )(p, cnt, selfterm, Wl)


def kernel(x, edge_index, W1l, b1l, W1r, W2l, b2l, W2r):
    n, feat = x.shape
    e = edge_index.shape[1]
    pad_rows = NS
    n_acc = -(-(n + pad_rows) // (NS * 8)) * (NS * 8)

    chunks = -(-e // (NW * K))
    e_pad = NW * chunks * K
    src = edge_index[0].astype(jnp.int32)
    dst = edge_index[1].astype(jnp.int32)
    if e_pad != e:
        pidx = jnp.arange(e_pad - e, dtype=jnp.int32)
        src = jnp.concatenate([src, pidx % n])
        dst = jnp.concatenate([dst, n + pidx % pad_rows])
    srcB = src.reshape(NW, chunks, K)
    dstB = dst.reshape(NW, chunks, K)

    ones = jnp.ones((K, feat), jnp.float32)
    zf = jnp.zeros((n_acc, feat), jnp.float32)

    agg = _sc_agg_kernel(n_acc, feat, chunks)
    cntk = _sc_cnt_kernel(n_acc, feat, chunks)

    cnt = cntk(dstB, ones, zf)
    self1 = _tc_self(x, W1r, b1l.reshape(1, feat))
    p1 = agg(x, srcB, dstB, zf)
    h = _tc_mix(p1, cnt, self1, W1l, relu=True)
    self2 = _tc_self(h, W2r, b2l.reshape(1, feat))
    p2 = agg(h, srcB, dstB, zf)
    z = _tc_mix(p2, cnt, self2, W2l, relu=False)
    return z
